# baseline (speedup 1.0000x reference)
import functools

import jax
import jax.numpy as jnp
from jax import lax
from jax.experimental import pallas as pl
from jax.experimental.pallas import tpu as pltpu
from jax.experimental.pallas import tpu_sc as plsc

BATCH = 8
N_CLS = 50
N_CTX = 12
D = 768
CTX_LEN = 77
N_TAIL = 64
MIX = 16
TILES_PER_B = 4
CLS_PER_TILE = 13


def _body(ctx_hbm, pre_hbm, suf_hbm, emb_hbm, tok_hbm, nl_hbm, out_hbm,
          idx_v, ens_v, ctx_v, nl_v, sem):
    nc = 2
    wid = lax.axis_index("s") * nc + lax.axis_index("c")
    b = wid // TILES_PER_B
    g = wid % TILES_PER_B
    i_start = g * CLS_PER_TILE
    i_count = jnp.minimum(CLS_PER_TILE, N_CLS - i_start)

    pltpu.sync_copy(tok_hbm.at[b], idx_v)
    pltpu.async_copy(emb_hbm.at[idx_v], ens_v, sem).wait()
    pltpu.sync_copy(ctx_hbm.at[b], ctx_v)
    pltpu.sync_copy(nl_hbm, nl_v)

    def per_class(j, carry):
        i = i_start + j
        n = b * N_CLS + i
        ell = nl_v[pl.ds(i, 16)][0]
        pltpu.sync_copy(pre_hbm.at[pl.ds(i, 1)], out_hbm.at[n, pl.ds(0, 1)])
        pltpu.sync_copy(ctx_v, out_hbm.at[n, pl.ds(1, N_CTX)])
        pltpu.sync_copy(suf_hbm.at[i, pl.ds(0, MIX)],
                        out_hbm.at[n, pl.ds(13, MIX)])
        pltpu.sync_copy(ens_v.at[pl.ds(0, MIX)],
                        out_hbm.at[n, pl.ds(13 + ell, MIX)])
        pltpu.sync_copy(ens_v.at[pl.ds(MIX - ell, N_TAIL - MIX)],
                        out_hbm.at[n, pl.ds(13 + MIX, N_TAIL - MIX)])
        return carry

    lax.fori_loop(0, i_count, per_class, 0)


def kernel(ctx, token_prefix, token_suffix, embed_table, tokenized_ensemble,
           name_lens, tokenized_prompts):
    tok64 = tokenized_ensemble[:, :N_TAIL]
    pre2d = token_prefix.reshape(N_CLS, D)
    nl64 = jnp.zeros((80,), jnp.int32).at[:N_CLS].set(name_lens)

    mesh = plsc.VectorSubcoreMesh(core_axis_name="c", subcore_axis_name="s")
    call = functools.partial(
        pl.kernel,
        mesh=mesh,
        compiler_params=pltpu.CompilerParams(use_tc_tiling_on_sc=False),
        out_type=jax.ShapeDtypeStruct((BATCH * N_CLS, CTX_LEN, D), jnp.float32),
        scratch_types=[
            pltpu.VMEM((N_TAIL,), jnp.int32),
            pltpu.VMEM((N_TAIL, D), jnp.float32),
            pltpu.VMEM((N_CTX, D), jnp.float32),
            pltpu.VMEM((80,), jnp.int32),
            pltpu.SemaphoreType.DMA,
        ],
    )(_body)
    prompts = call(ctx, pre2d, token_suffix, embed_table, tok64, nl64)

    tp = jnp.broadcast_to(tokenized_prompts[None],
                          (BATCH, N_CLS, CTX_LEN)).reshape(BATCH * N_CLS, CTX_LEN)
    return (prompts, tp)


# speedup vs baseline: 1.0010x; 1.0010x over previous
import functools

import jax
import jax.numpy as jnp
from jax import lax
from jax.experimental import pallas as pl
from jax.experimental.pallas import tpu as pltpu
from jax.experimental.pallas import tpu_sc as plsc

BATCH = 8
N_CLS = 50
N_CTX = 12
D = 768
CTX_LEN = 77
N_TAIL = 64
MIX = 16
TILES_PER_B = 4
CLS_PER_TILE = 13


def _body(ctx_hbm, pre_hbm, suf_hbm, emb_hbm, tok_hbm, nl_hbm, out_hbm,
          idx_v, ens_v, ctx_v, nl_v, sem, sem_a, sem_w):
    nc = 2
    wid = lax.axis_index("s") * nc + lax.axis_index("c")
    b = wid // TILES_PER_B
    g = wid % TILES_PER_B
    i_start = g * CLS_PER_TILE
    i_count = jnp.minimum(CLS_PER_TILE, N_CLS - i_start)

    pltpu.sync_copy(tok_hbm.at[b], idx_v)
    pltpu.async_copy(emb_hbm.at[idx_v], ens_v, sem).wait()
    pltpu.sync_copy(ctx_hbm.at[b], ctx_v)
    pltpu.sync_copy(nl_hbm, nl_v)

    def suf_desc(j):
        i = i_start + j
        n = b * N_CLS + i
        return pltpu.make_async_copy(
            suf_hbm.at[i, pl.ds(0, MIX)], out_hbm.at[n, pl.ds(13, MIX)], sem_a)

    suf_desc(0).start()

    def per_class(j, carry):
        i = i_start + j
        n = b * N_CLS + i
        ell = nl_v[pl.ds(i, 16)][0]
        suf_desc(j).wait()
        pltpu.make_async_copy(
            pre_hbm.at[pl.ds(i, 1)], out_hbm.at[n, pl.ds(0, 1)], sem_w).start()
        pltpu.make_async_copy(
            ctx_v, out_hbm.at[n, pl.ds(1, N_CTX)], sem_w).start()
        pltpu.make_async_copy(
            ens_v.at[pl.ds(0, MIX)],
            out_hbm.at[n, pl.ds(13 + ell, MIX)], sem_w).start()
        pltpu.make_async_copy(
            ens_v.at[pl.ds(MIX - ell, N_TAIL - MIX)],
            out_hbm.at[n, pl.ds(13 + MIX, N_TAIL - MIX)], sem_w).start()

        @pl.when(j + 1 < i_count)
        def _():
            suf_desc(j + 1).start()

        return carry

    lax.fori_loop(0, i_count, per_class, 0)

    def drain(j, carry):
        n = b * N_CLS + i_start + j
        pltpu.make_async_copy(
            pre_hbm.at[pl.ds(0, 1)], out_hbm.at[n, pl.ds(0, 1)], sem_w).wait()
        pltpu.make_async_copy(
            ctx_v, out_hbm.at[n, pl.ds(1, N_CTX)], sem_w).wait()
        pltpu.make_async_copy(
            ens_v.at[pl.ds(0, MIX)],
            out_hbm.at[n, pl.ds(13, MIX)], sem_w).wait()
        pltpu.make_async_copy(
            ens_v.at[pl.ds(0, N_TAIL - MIX)],
            out_hbm.at[n, pl.ds(13 + MIX, N_TAIL - MIX)], sem_w).wait()
        return carry

    lax.fori_loop(0, i_count, drain, 0)


def kernel(ctx, token_prefix, token_suffix, embed_table, tokenized_ensemble,
           name_lens, tokenized_prompts):
    tok64 = tokenized_ensemble[:, :N_TAIL]
    pre2d = token_prefix.reshape(N_CLS, D)
    nl64 = jnp.zeros((80,), jnp.int32).at[:N_CLS].set(name_lens)

    mesh = plsc.VectorSubcoreMesh(core_axis_name="c", subcore_axis_name="s")
    call = functools.partial(
        pl.kernel,
        mesh=mesh,
        compiler_params=pltpu.CompilerParams(use_tc_tiling_on_sc=False),
        out_type=jax.ShapeDtypeStruct((BATCH * N_CLS, CTX_LEN, D), jnp.float32),
        scratch_types=[
            pltpu.VMEM((N_TAIL,), jnp.int32),
            pltpu.VMEM((N_TAIL, D), jnp.float32),
            pltpu.VMEM((N_CTX, D), jnp.float32),
            pltpu.VMEM((80,), jnp.int32),
            pltpu.SemaphoreType.DMA,
            pltpu.SemaphoreType.DMA,
            pltpu.SemaphoreType.DMA,
        ],
    )(_body)
    prompts = call(ctx, pre2d, token_suffix, embed_table, tok64, nl64)

    tp = jnp.broadcast_to(tokenized_prompts[None],
                          (BATCH, N_CLS, CTX_LEN)).reshape(BATCH * N_CLS, CTX_LEN)
    return (prompts, tp)


# speedup vs baseline: 1.0036x; 1.0026x over previous
import functools

import jax
import jax.numpy as jnp
from jax import lax
from jax.experimental import pallas as pl
from jax.experimental.pallas import tpu as pltpu
from jax.experimental.pallas import tpu_sc as plsc

BATCH = 8
N_CLS = 50
N_CTX = 12
D = 768
CTX_LEN = 77
N_TAIL = 64
MIX = 16
TILES_PER_B = 4
CLS_PER_TILE = 13


def _body(ctx_hbm, pre_hbm, suf_hbm, emb_hbm, tok_hbm, nl_hbm, out_hbm,
          idx_v, ens_v, ctx_v, nl_v, sem, sem_a, sem_w):
    nc = 2
    wid = lax.axis_index("s") * nc + lax.axis_index("c")
    b = wid // TILES_PER_B
    g = wid % TILES_PER_B
    i_start = g * CLS_PER_TILE
    i_count = jnp.minimum(CLS_PER_TILE, N_CLS - i_start)

    pltpu.sync_copy(tok_hbm.at[b], idx_v)
    pltpu.async_copy(emb_hbm.at[idx_v], ens_v, sem).wait()
    pltpu.sync_copy(ctx_hbm.at[b], ctx_v)
    pltpu.sync_copy(nl_hbm, nl_v)

    def suf_desc(j):
        i = i_start + j
        n = b * N_CLS + i
        return pltpu.make_async_copy(
            suf_hbm.at[i, pl.ds(0, MIX)], out_hbm.at[n, pl.ds(13, MIX)], sem_a)

    def start_suf(j, carry):
        suf_desc(j).start()
        return carry

    lax.fori_loop(0, i_count, start_suf, 0)

    def per_class(j, carry):
        i = i_start + j
        n = b * N_CLS + i
        ell = nl_v[pl.ds(i, 16)][0]
        suf_desc(j).wait()
        pltpu.make_async_copy(
            pre_hbm.at[pl.ds(i, 1)], out_hbm.at[n, pl.ds(0, 1)], sem_w).start()
        pltpu.make_async_copy(
            ctx_v, out_hbm.at[n, pl.ds(1, N_CTX)], sem_w).start()
        pltpu.make_async_copy(
            ens_v.at[pl.ds(0, MIX)],
            out_hbm.at[n, pl.ds(13 + ell, MIX)], sem_w).start()
        pltpu.make_async_copy(
            ens_v.at[pl.ds(MIX - ell, N_TAIL - MIX)],
            out_hbm.at[n, pl.ds(13 + MIX, N_TAIL - MIX)], sem_w).start()
        return carry

    lax.fori_loop(0, i_count, per_class, 0)

    def drain(j, carry):
        n = b * N_CLS + i_start + j
        pltpu.make_async_copy(
            pre_hbm.at[pl.ds(0, 1)], out_hbm.at[n, pl.ds(0, 1)], sem_w).wait()
        pltpu.make_async_copy(
            ctx_v, out_hbm.at[n, pl.ds(1, N_CTX)], sem_w).wait()
        pltpu.make_async_copy(
            ens_v.at[pl.ds(0, MIX)],
            out_hbm.at[n, pl.ds(13, MIX)], sem_w).wait()
        pltpu.make_async_copy(
            ens_v.at[pl.ds(0, N_TAIL - MIX)],
            out_hbm.at[n, pl.ds(13 + MIX, N_TAIL - MIX)], sem_w).wait()
        return carry

    lax.fori_loop(0, i_count, drain, 0)


def kernel(ctx, token_prefix, token_suffix, embed_table, tokenized_ensemble,
           name_lens, tokenized_prompts):
    tok64 = tokenized_ensemble[:, :N_TAIL]
    pre2d = token_prefix.reshape(N_CLS, D)
    nl64 = jnp.zeros((80,), jnp.int32).at[:N_CLS].set(name_lens)

    mesh = plsc.VectorSubcoreMesh(core_axis_name="c", subcore_axis_name="s")
    call = functools.partial(
        pl.kernel,
        mesh=mesh,
        compiler_params=pltpu.CompilerParams(use_tc_tiling_on_sc=False),
        out_type=jax.ShapeDtypeStruct((BATCH * N_CLS, CTX_LEN, D), jnp.float32),
        scratch_types=[
            pltpu.VMEM((N_TAIL,), jnp.int32),
            pltpu.VMEM((N_TAIL, D), jnp.float32),
            pltpu.VMEM((N_CTX, D), jnp.float32),
            pltpu.VMEM((80,), jnp.int32),
            pltpu.SemaphoreType.DMA,
            pltpu.SemaphoreType.DMA,
            pltpu.SemaphoreType.DMA,
        ],
    )(_body)
    prompts = call(ctx, pre2d, token_suffix, embed_table, tok64, nl64)

    tp = jnp.broadcast_to(tokenized_prompts[None],
                          (BATCH, N_CLS, CTX_LEN)).reshape(BATCH * N_CLS, CTX_LEN)
    return (prompts, tp)


# speedup vs baseline: 2.4951x; 2.4862x over previous
import functools

import jax
import jax.numpy as jnp
from jax import lax
from jax.experimental import pallas as pl
from jax.experimental.pallas import tpu as pltpu
from jax.experimental.pallas import tpu_sc as plsc

BATCH = 8
N_CLS = 50
N_CTX = 12
D = 768
CTX_LEN = 77
N_TAIL = 64
MIX = 16
TILES_PER_B = 4
CLS_PER_TILE = 13


def _body(ctx_hbm, pre_hbm, suf_hbm, emb_hbm, tok_hbm, nl_hbm, out_hbm,
          idx_v, ens_v, ctx_v, nl_v, pre_v, suf_v, sem, sem_a, sem_w, sem_s):
    nc = 2
    wid = lax.axis_index("s") * nc + lax.axis_index("c")
    b = wid // TILES_PER_B
    g = wid % TILES_PER_B
    i_start = g * CLS_PER_TILE
    i_count = jnp.minimum(CLS_PER_TILE, N_CLS - i_start)

    pltpu.sync_copy(tok_hbm.at[b], idx_v)
    pltpu.async_copy(emb_hbm.at[idx_v], ens_v, sem).wait()
    pltpu.sync_copy(ctx_hbm.at[b], ctx_v)
    pltpu.sync_copy(pre_hbm.at[pl.ds(i_start, CLS_PER_TILE)], pre_v)
    pltpu.sync_copy(nl_hbm, nl_v)

    def suf_fetch(j):
        return pltpu.make_async_copy(
            suf_hbm.at[i_start + j, pl.ds(0, MIX)], suf_v.at[j % 2], sem_s)

    def suf_write(j):
        n = b * N_CLS + i_start + j
        return pltpu.make_async_copy(
            suf_v.at[j % 2], out_hbm.at[n, pl.ds(13, MIX)], sem_a)

    def overlay_write(j):
        i = i_start + j
        n = b * N_CLS + i
        ell = nl_v[pl.ds(i, 16)][0]
        pltpu.make_async_copy(
            ens_v.at[pl.ds(0, MIX)],
            out_hbm.at[n, pl.ds(13 + ell, MIX)], sem_w).start()

    suf_fetch(0).start()

    def per_class(j, carry):
        i = i_start + j
        n = b * N_CLS + i
        ell = nl_v[pl.ds(i, 16)][0]
        suf_fetch(j).wait()
        pltpu.make_async_copy(
            pre_v.at[pl.ds(j, 1)], out_hbm.at[n, pl.ds(0, 1)], sem_w).start()
        pltpu.make_async_copy(
            ctx_v, out_hbm.at[n, pl.ds(1, N_CTX)], sem_w).start()
        suf_write(j).start()
        pltpu.make_async_copy(
            ens_v.at[pl.ds(MIX - ell, N_TAIL - MIX)],
            out_hbm.at[n, pl.ds(13 + MIX, N_TAIL - MIX)], sem_w).start()

        @pl.when(j >= 1)
        def _():
            suf_write(j - 1).wait()
            overlay_write(j - 1)

        @pl.when(j + 1 < i_count)
        def _():
            suf_fetch(j + 1).start()

        return carry

    lax.fori_loop(0, i_count, per_class, 0)
    suf_write(i_count - 1).wait()
    overlay_write(i_count - 1)

    def drain(j, carry):
        n = b * N_CLS + i_start + j
        pltpu.make_async_copy(
            pre_v.at[pl.ds(0, 1)], out_hbm.at[n, pl.ds(0, 1)], sem_w).wait()
        pltpu.make_async_copy(
            ctx_v, out_hbm.at[n, pl.ds(1, N_CTX)], sem_w).wait()
        pltpu.make_async_copy(
            ens_v.at[pl.ds(0, MIX)],
            out_hbm.at[n, pl.ds(13, MIX)], sem_w).wait()
        pltpu.make_async_copy(
            ens_v.at[pl.ds(0, N_TAIL - MIX)],
            out_hbm.at[n, pl.ds(13 + MIX, N_TAIL - MIX)], sem_w).wait()
        return carry

    lax.fori_loop(0, i_count, drain, 0)


def kernel(ctx, token_prefix, token_suffix, embed_table, tokenized_ensemble,
           name_lens, tokenized_prompts):
    tok64 = tokenized_ensemble[:, :N_TAIL]
    pre2d = jnp.zeros((N_CLS + CLS_PER_TILE, D), jnp.float32)
    pre2d = pre2d.at[:N_CLS].set(token_prefix.reshape(N_CLS, D))
    nl64 = jnp.zeros((80,), jnp.int32).at[:N_CLS].set(name_lens)

    mesh = plsc.VectorSubcoreMesh(core_axis_name="c", subcore_axis_name="s")
    call = functools.partial(
        pl.kernel,
        mesh=mesh,
        compiler_params=pltpu.CompilerParams(use_tc_tiling_on_sc=False),
        out_type=jax.ShapeDtypeStruct((BATCH * N_CLS, CTX_LEN, D), jnp.float32),
        scratch_types=[
            pltpu.VMEM((N_TAIL,), jnp.int32),
            pltpu.VMEM((N_TAIL, D), jnp.float32),
            pltpu.VMEM((N_CTX, D), jnp.float32),
            pltpu.VMEM((80,), jnp.int32),
            pltpu.VMEM((CLS_PER_TILE, D), jnp.float32),
            pltpu.VMEM((2, MIX, D), jnp.float32),
            pltpu.SemaphoreType.DMA,
            pltpu.SemaphoreType.DMA,
            pltpu.SemaphoreType.DMA,
            pltpu.SemaphoreType.DMA,
        ],
    )(_body)
    prompts = call(ctx, pre2d, token_suffix, embed_table, tok64, nl64)

    tp = jnp.broadcast_to(tokenized_prompts[None],
                          (BATCH, N_CLS, CTX_LEN)).reshape(BATCH * N_CLS, CTX_LEN)
    return (prompts, tp)
